# baseline (speedup 1.0000x reference)
import jax
import jax.numpy as jnp
from jax.experimental import pallas as pl
from jax.experimental.pallas import tpu as pltpu

C_CONST = 1.0
GAMMA = 1.0
GATE_EPS = 1e-9
LOG2E = 1.4426950408889634
LN2 = 0.6931471805599453
SQ = LOG2E ** 0.5
C2 = C_CONST * LOG2E
EPS2 = GATE_EPS * LOG2E * LOG2E
NEG_BIG = -1e30

B_TOT = 64
T_LEN = 512
N_CORES = 2
B_BLK = B_TOT // N_CORES
N_GRP = 4
U_SUB = 128


def _ladd2(a, b):
    mx = jnp.maximum(a, b)
    mn = jnp.minimum(a, b)
    return mx + jnp.log2(1.0 + jnp.exp2(mn - mx))


def _nsm2(aa, bb):
    mn = jnp.minimum(aa, bb)
    mx = jnp.maximum(aa, bb)
    return mn - jnp.log2(1.0 + jnp.exp2(mn - mx))


def _trans2(aa, a, b):
    u = a * b
    q = u * jax.lax.rsqrt(u * u + EPS2)
    h = 0.5 * _nsm2(aa, b * b)
    return C2 + h + q * h


def _sshift(v, s, fill):
    pad = jnp.full((s, v.shape[1]), fill, v.dtype)
    return jnp.concatenate([pad, v[:-s]], axis=0)


def _gshift(v, k, fill):
    pad = jnp.full((v.shape[0], B_BLK * k), fill, v.dtype)
    return jnp.concatenate([pad, v[:, : -B_BLK * k]], axis=1)


def _prefix_sum_pk(v):
    for s in (1, 2, 4, 8, 16, 32, 64):
        v = v + _sshift(v, s, 0.0)
    tot = v[U_SUB - 1 : U_SUB, :]
    excl = _gshift(tot, 1, 0.0) + _gshift(tot, 2, 0.0) + _gshift(tot, 3, 0.0)
    return v + excl


TINY = 1e-30


def _prefix_lse2_pk(v):
    v3 = v.reshape(U_SUB // 8, 8, N_GRP * B_BLK)
    m8 = jnp.max(v3, axis=1, keepdims=True)
    p = jnp.exp2(v3 - m8)
    for s in (1, 2, 4):
        pad = jnp.zeros((U_SUB // 8, s, N_GRP * B_BLK), v.dtype)
        p = p + jnp.concatenate([pad, p[:, :-s, :]], axis=1)
    t8 = (m8 + jnp.log2(p[:, 7:8, :] + TINY)).reshape(U_SUB // 8, -1)
    for s in (1, 2, 4, 8):
        t8 = _ladd2(t8, _sshift(t8, s, NEG_BIG))
    tot = t8[U_SUB // 8 - 1 :, :]
    s1 = _gshift(tot, 1, NEG_BIG)
    s2 = _gshift(tot, 2, NEG_BIG)
    s3 = _gshift(tot, 3, NEG_BIG)
    excl = _ladd2(_ladd2(s1, s2), s3)
    e8 = _sshift(t8, 1, NEG_BIG).reshape(U_SUB // 8, 1, -1)
    eg = excl.reshape(1, 1, N_GRP * B_BLK)
    m3 = jnp.maximum(jnp.maximum(m8, e8), eg)
    s = m3 + jnp.log2(
        p * jnp.exp2(m8 - m3) + jnp.exp2(e8 - m3) + jnp.exp2(eg - m3) + TINY
    )
    return s.reshape(U_SUB, N_GRP * B_BLK), excl


def _msm_body(xr_ref, y_ref, o_ref):
    y2 = y_ref[0] * SQ
    x0 = xr_ref[0, 0][0:1, :] * SQ
    ysh = jnp.concatenate(
        [_gshift(y2[U_SUB - 1 : U_SUB, :], 1, 0.0), y2[:-1]], axis=0
    )
    a_l = y2 - ysh
    aa_l = a_l * a_l
    sub = jax.lax.broadcasted_iota(jnp.int32, y2.shape, 0)
    lanes = jax.lax.broadcasted_iota(jnp.int32, y2.shape, 1)
    col0 = (sub == 0) & (lanes < B_BLK)

    def precompute(xn, xv):
        m2 = (xn - y2) ** 2
        d = xn - xv
        u2 = _trans2(d * d, d, xn - y2)
        l2 = _trans2(aa_l, a_l, y2 - xn)
        cl = _prefix_sum_pk(jnp.where(col0, 0.0, l2))
        clb = _gshift(cl[U_SUB - 1 : U_SUB, :], 1, 0.0)
        return m2 - cl, u2 - cl, cl, clb

    left0 = _trans2(aa_l, a_l, y2 - x0)
    v0 = jnp.where(col0, (x0 - y2[0:1, :]) ** 2, left0)
    r = -_prefix_sum_pk(v0)
    bnd = _gshift(r[U_SUB - 1 : U_SUB, :], 1, NEG_BIG)

    x1 = xr_ref[1, 0][0:1, :] * SQ
    carry0 = (r, bnd, x1) + precompute(x1, x0)

    def body(i, carry):
        r, bnd, xv, m2c, u2c, cl, clb = carry
        rsh = jnp.concatenate([bnd, r[:-1]], axis=0)
        t = _ladd2(rsh - m2c, r - u2c)
        s, excl = _prefix_lse2_pk(t)
        r_new = s - cl
        bnd_new = excl - clb
        nxt = jnp.minimum(i + 1, T_LEN - 1)
        xn = xr_ref[pl.ds(nxt, 1), 0].reshape(8, N_GRP * B_BLK)[0:1, :] * SQ
        return (r_new, bnd_new, xn) + precompute(xn, xv)

    r = jax.lax.fori_loop(1, T_LEN, body, carry0, unroll=4)[0]
    last = r[U_SUB - 1 : U_SUB, :]
    gmask = lanes[0:1, :] >= (N_GRP - 1) * B_BLK
    total = -LN2 * jnp.sum(jnp.where(gmask, last, 0.0))
    o_ref[...] = jnp.full(o_ref.shape, total, jnp.float32)


def kernel(x, y):
    xb = x[:, 0, :]
    yb = y[:, 0, :]
    xt = jnp.transpose(xb).reshape(T_LEN, N_CORES, 1, 1, B_BLK)
    xrow = jnp.broadcast_to(
        xt, (T_LEN, N_CORES, 8, N_GRP, B_BLK)
    ).reshape(T_LEN, N_CORES, 8, N_GRP * B_BLK)
    ypk = jnp.transpose(
        yb.reshape(N_CORES, B_BLK, N_GRP, U_SUB), (0, 3, 2, 1)
    ).reshape(N_CORES, U_SUB, N_GRP * B_BLK)
    partial = pl.pallas_call(
        _msm_body,
        grid=(N_CORES,),
        in_specs=[
            pl.BlockSpec((T_LEN, 1, 8, N_GRP * B_BLK), lambda p: (0, p, 0, 0)),
            pl.BlockSpec((1, U_SUB, N_GRP * B_BLK), lambda p: (p, 0, 0)),
        ],
        out_specs=pl.BlockSpec((1, 8, 128), lambda p: (p, 0, 0)),
        out_shape=jax.ShapeDtypeStruct((N_CORES, 8, 128), jnp.float32),
        compiler_params=pltpu.CompilerParams(
            dimension_semantics=("parallel",),
        ),
    )(xrow, ypk)
    return jnp.sum(partial[:, 0, 0]) / B_TOT


# speedup vs baseline: 1.0644x; 1.0644x over previous
import jax
import jax.numpy as jnp
from jax.experimental import pallas as pl
from jax.experimental.pallas import tpu as pltpu

C_CONST = 1.0
GAMMA = 1.0
GATE_EPS = 1e-9
LOG2E = 1.4426950408889634
LN2 = 0.6931471805599453
SQ = LOG2E ** 0.5
C2 = C_CONST * LOG2E
EPS2 = GATE_EPS * LOG2E * LOG2E
NEG_BIG = -1e30

B_TOT = 64
T_LEN = 512
N_CORES = 2
B_BLK = B_TOT // N_CORES
N_GRP = 4
U_SUB = 128


def _ladd2(a, b):
    mx = jnp.maximum(a, b)
    mn = jnp.minimum(a, b)
    return mx + jnp.log2(1.0 + jnp.exp2(mn - mx))


def _nsm2(aa, bb):
    mn = jnp.minimum(aa, bb)
    mx = jnp.maximum(aa, bb)
    return mn - jnp.log2(1.0 + jnp.exp2(mn - mx))


def _trans2(aa, a, b):
    u = a * b
    q = u * jax.lax.rsqrt(u * u + EPS2)
    h = 0.5 * _nsm2(aa, b * b)
    return C2 + h + q * h


def _sshift(v, s, fill):
    pad = jnp.full((s, v.shape[1]), fill, v.dtype)
    return jnp.concatenate([pad, v[:-s]], axis=0)


def _gshift(v, k, fill):
    pad = jnp.full((v.shape[0], B_BLK * k), fill, v.dtype)
    return jnp.concatenate([pad, v[:, : -B_BLK * k]], axis=1)


def _prefix_sum_pk(v):
    for s in (1, 2, 4, 8, 16, 32, 64):
        v = v + _sshift(v, s, 0.0)
    tot = v[U_SUB - 1 : U_SUB, :]
    excl = _gshift(tot, 1, 0.0) + _gshift(tot, 2, 0.0) + _gshift(tot, 3, 0.0)
    return v + excl


def _prefix_lse2_pk(v):
    for s in (1, 2, 4, 8, 16, 32, 64):
        v = _ladd2(v, _sshift(v, s, NEG_BIG))
    tot = v[U_SUB - 1 : U_SUB, :]
    s1 = _gshift(tot, 1, NEG_BIG)
    s2 = _gshift(tot, 2, NEG_BIG)
    s3 = _gshift(tot, 3, NEG_BIG)
    excl = _ladd2(_ladd2(s1, s2), s3)
    return _ladd2(v, excl), excl


def _msm_body(xr_ref, y_ref, o_ref):
    y2 = y_ref[0] * SQ
    x0 = xr_ref[0, 0][0:1, :] * SQ
    ysh = jnp.concatenate(
        [_gshift(y2[U_SUB - 1 : U_SUB, :], 1, 0.0), y2[:-1]], axis=0
    )
    a_l = y2 - ysh
    aa_l = a_l * a_l
    sub = jax.lax.broadcasted_iota(jnp.int32, y2.shape, 0)
    lanes = jax.lax.broadcasted_iota(jnp.int32, y2.shape, 1)
    col0 = (sub == 0) & (lanes < B_BLK)

    def precompute(xn, xv):
        m2 = (xn - y2) ** 2
        d = xn - xv
        u2 = _trans2(d * d, d, xn - y2)
        l2 = _trans2(aa_l, a_l, y2 - xn)
        cl = _prefix_sum_pk(jnp.where(col0, 0.0, l2))
        clb = _gshift(cl[U_SUB - 1 : U_SUB, :], 1, 0.0)
        return m2 - cl, u2 - cl, cl, clb

    left0 = _trans2(aa_l, a_l, y2 - x0)
    v0 = jnp.where(col0, (x0 - y2[0:1, :]) ** 2, left0)
    r = -_prefix_sum_pk(v0)
    bnd = _gshift(r[U_SUB - 1 : U_SUB, :], 1, NEG_BIG)

    x1 = xr_ref[1, 0][0:1, :] * SQ
    carry0 = (r, bnd, x1) + precompute(x1, x0)

    def body(i, carry):
        r, bnd, xv, m2c, u2c, cl, clb = carry
        rsh = jnp.concatenate([bnd, r[:-1]], axis=0)
        t = _ladd2(rsh - m2c, r - u2c)
        s, excl = _prefix_lse2_pk(t)
        r_new = s - cl
        bnd_new = excl - clb
        nxt = jnp.minimum(i + 1, T_LEN - 1)
        xn = xr_ref[pl.ds(nxt, 1), 0].reshape(8, N_GRP * B_BLK)[0:1, :] * SQ
        return (r_new, bnd_new, xn) + precompute(xn, xv)

    r = jax.lax.fori_loop(1, T_LEN, body, carry0, unroll=8)[0]
    last = r[U_SUB - 1 : U_SUB, :]
    gmask = lanes[0:1, :] >= (N_GRP - 1) * B_BLK
    total = -LN2 * jnp.sum(jnp.where(gmask, last, 0.0))
    o_ref[...] = jnp.full(o_ref.shape, total, jnp.float32)


def kernel(x, y):
    xb = x[:, 0, :]
    yb = y[:, 0, :]
    xt = jnp.transpose(xb).reshape(T_LEN, N_CORES, 1, 1, B_BLK)
    xrow = jnp.broadcast_to(
        xt, (T_LEN, N_CORES, 8, N_GRP, B_BLK)
    ).reshape(T_LEN, N_CORES, 8, N_GRP * B_BLK)
    ypk = jnp.transpose(
        yb.reshape(N_CORES, B_BLK, N_GRP, U_SUB), (0, 3, 2, 1)
    ).reshape(N_CORES, U_SUB, N_GRP * B_BLK)
    partial = pl.pallas_call(
        _msm_body,
        grid=(N_CORES,),
        in_specs=[
            pl.BlockSpec((T_LEN, 1, 8, N_GRP * B_BLK), lambda p: (0, p, 0, 0)),
            pl.BlockSpec((1, U_SUB, N_GRP * B_BLK), lambda p: (p, 0, 0)),
        ],
        out_specs=pl.BlockSpec((1, 8, 128), lambda p: (p, 0, 0)),
        out_shape=jax.ShapeDtypeStruct((N_CORES, 8, 128), jnp.float32),
        compiler_params=pltpu.CompilerParams(
            dimension_semantics=("parallel",),
        ),
    )(xrow, ypk)
    return jnp.sum(partial[:, 0, 0]) / B_TOT
